# Initial kernel scaffold; baseline (speedup 1.0000x reference)
import jax
import jax.numpy as jnp
from jax.experimental import pallas as pl


def kernel(positions, neighbors):
    raise NotImplementedError("write your pallas kernel here")


# speedup vs baseline: 192.2860x; 192.2860x over previous
import functools

import jax
import jax.numpy as jnp
from jax import lax
from jax.experimental import pallas as pl
from jax.experimental.pallas import tpu as pltpu
from jax.experimental.pallas import tpu_sc as plsc

L = 16


def _rsqrt_nr(s):
    i = plsc.bitcast(s, jnp.int32)
    i = jnp.int32(0x5F375A86) - (i >> 1)
    y = plsc.bitcast(i, jnp.float32)
    xh = s * jnp.float32(0.5)
    y = y * (jnp.float32(1.5) - xh * y * y)
    y = y * (jnp.float32(1.5) - xh * y * y)
    return y


def _make_sc_kernel(B, N, K, WPB, CA):
    SLAB = N // WPB
    NCHUNK = SLAB // CA
    KV = K // L
    CAE = CA * K
    info = plsc.get_sparse_core_info()
    NC = info.num_cores

    mesh = plsc.VectorSubcoreMesh(core_axis_name="c", subcore_axis_name="s")

    @functools.partial(
        pl.kernel,
        mesh=mesh,
        out_type=jax.ShapeDtypeStruct((B, N * K), jnp.float32),
        scratch_types=[
            pltpu.VMEM((N * 3,), jnp.float32),
            pltpu.VMEM((CAE,), jnp.int32),
            pltpu.VMEM((CAE,), jnp.float32),
        ],
        compiler_params=pltpu.CompilerParams(
            use_tc_tiling_on_sc=False, needs_layout_passes=False
        ),
    )
    def k(pos_hbm, nbr_hbm, out_hbm, table_v, nbr_v, out_v):
        wid = lax.axis_index("s") * NC + lax.axis_index("c")
        b = wid // WPB
        base_e = (wid % WPB) * SLAB * K
        base_a = (wid % WPB) * SLAB

        pltpu.sync_copy(pos_hbm.at[b], table_v)

        def chunk_body(c, carry):
            e0 = base_e + c * CAE
            a0 = base_a + c * CA
            pltpu.sync_copy(nbr_hbm.at[b, pl.ds(e0, CAE)], nbr_v)

            def atom_body(a, carry2):
                g3 = (a0 + a) * 3
                cidx = jnp.full((L,), g3, jnp.int32)
                cx = plsc.load_gather(table_v, [cidx])
                cy = plsc.load_gather(table_v, [cidx + 1])
                cz = plsc.load_gather(table_v, [cidx + 2])
                for kv in range(KV):
                    nbr3 = nbr_v[pl.ds(a * K + kv * L, L)] * 3
                    xg = plsc.load_gather(table_v, [nbr3])
                    yg = plsc.load_gather(table_v, [nbr3 + 1])
                    zg = plsc.load_gather(table_v, [nbr3 + 2])
                    dx = xg - cx
                    dy = yg - cy
                    dz = zg - cz
                    s = dx * dx + dy * dy + dz * dz
                    d = s * _rsqrt_nr(s)
                    out_v[pl.ds(a * K + kv * L, L)] = d
                return carry2

            lax.fori_loop(0, CA, atom_body, 0)
            pltpu.sync_copy(out_v, out_hbm.at[b, pl.ds(e0, CAE)])
            return carry

        lax.fori_loop(0, NCHUNK, chunk_body, 0)

    return k


def kernel(positions, neighbors):
    B, N, _ = positions.shape
    K = neighbors.shape[2]
    k = _make_sc_kernel(B, N, K, WPB=8, CA=125)
    out = k(positions.reshape(B, N * 3), neighbors.reshape(B, N * K))
    return out.reshape(B, N, K)


# speedup vs baseline: 266.0939x; 1.3838x over previous
import functools

import jax
import jax.numpy as jnp
from jax import lax
from jax.experimental import pallas as pl
from jax.experimental.pallas import tpu as pltpu
from jax.experimental.pallas import tpu_sc as plsc

L = 16
U = 8


def _make_sc_kernel(B, N, K, WPB, CA):
    SLAB = N // WPB
    NCHUNK = SLAB // CA
    CAE = CA * K
    NG = (CA + L - 1) // L
    KB = K // U
    info = plsc.get_sparse_core_info()
    NC = info.num_cores

    mesh = plsc.VectorSubcoreMesh(core_axis_name="c", subcore_axis_name="s")

    @functools.partial(
        pl.kernel,
        mesh=mesh,
        out_type=jax.ShapeDtypeStruct((B, N * K), jnp.float32),
        scratch_types=[
            pltpu.VMEM((N,), jnp.float32),
            pltpu.VMEM((N,), jnp.float32),
            pltpu.VMEM((N,), jnp.float32),
            pltpu.VMEM((2, CAE), jnp.int32),
            pltpu.VMEM((2, CAE), jnp.float32),
            pltpu.SemaphoreType.DMA((2,)),
            pltpu.SemaphoreType.DMA((2,)),
        ],
        compiler_params=pltpu.CompilerParams(
            use_tc_tiling_on_sc=False, needs_layout_passes=False
        ),
    )
    def k(xs_hbm, ys_hbm, zs_hbm, nbr_hbm, out_hbm,
          xs_v, ys_v, zs_v, nbr_v, out_v, in_sem, out_sem):
        wid = lax.axis_index("s") * NC + lax.axis_index("c")
        b = wid // WPB
        base_a = (wid % WPB) * SLAB
        base_e = base_a * K

        pltpu.sync_copy(xs_hbm.at[b], xs_v)
        pltpu.sync_copy(ys_hbm.at[b], ys_v)
        pltpu.sync_copy(zs_hbm.at[b], zs_v)

        iota = lax.iota(jnp.int32, L)

        def in_copy(c, buf):
            e0 = base_e + c * CAE
            return pltpu.make_async_copy(
                nbr_hbm.at[b, pl.ds(e0, CAE)], nbr_v.at[buf], in_sem.at[buf])

        def out_copy(c, buf):
            e0 = base_e + c * CAE
            return pltpu.make_async_copy(
                out_v.at[buf], out_hbm.at[b, pl.ds(e0, CAE)], out_sem.at[buf])

        in_copy(0, 0).start()

        def chunk_body(c, carry):
            buf = lax.rem(c, 2)

            @pl.when(c >= 2)
            def _():
                out_copy(c - 2, buf).wait()

            in_copy(c, buf).wait()

            @pl.when(c + 1 < NCHUNK)
            def _():
                in_copy(c + 1, 1 - buf).start()

            bufv = jnp.full((L,), buf, jnp.int32)
            a0c = base_a + c * CA

            def group_body(g, carry2):
                al = jnp.minimum(g * L + iota, CA - 1)
                ag = a0c + al
                cx = plsc.load_gather(xs_v, [ag])
                cy = plsc.load_gather(ys_v, [ag])
                cz = plsc.load_gather(zs_v, [ag])
                ebase = al * K

                def kb_body(kb, carry3):
                    k0 = kb * U
                    R = range(U)
                    idx = [ebase + (k0 + j) for j in R]
                    nb = [plsc.load_gather(nbr_v, [bufv, idx[j]]) for j in R]
                    xg = [plsc.load_gather(xs_v, [nb[j]]) for j in R]
                    yg = [plsc.load_gather(ys_v, [nb[j]]) for j in R]
                    zg = [plsc.load_gather(zs_v, [nb[j]]) for j in R]
                    dx = [xg[j] - cx for j in R]
                    dy = [yg[j] - cy for j in R]
                    dz = [zg[j] - cz for j in R]
                    s = [dx[j] * dx[j] for j in R]
                    s = [s[j] + dy[j] * dy[j] for j in R]
                    s = [s[j] + dz[j] * dz[j] for j in R]
                    iv = [plsc.bitcast(s[j], jnp.int32) for j in R]
                    iv = [jnp.int32(0x5F375A86) - (iv[j] >> 1) for j in R]
                    y = [plsc.bitcast(iv[j], jnp.float32) for j in R]
                    xh = [s[j] * jnp.float32(0.5) for j in R]
                    t = [y[j] * y[j] for j in R]
                    t = [xh[j] * t[j] for j in R]
                    t = [jnp.float32(1.5) - t[j] for j in R]
                    y = [y[j] * t[j] for j in R]
                    d = [s[j] * y[j] for j in R]
                    for j in R:
                        plsc.store_scatter(out_v, [bufv, idx[j]], d[j])
                    return carry3

                lax.fori_loop(0, KB, kb_body, 0)
                return carry2

            lax.fori_loop(0, NG, group_body, 0)
            out_copy(c, buf).start()
            return carry

        lax.fori_loop(0, NCHUNK, chunk_body, 0)
        out_copy(NCHUNK - 2, (NCHUNK - 2) % 2).wait()
        out_copy(NCHUNK - 1, (NCHUNK - 1) % 2).wait()

    return k


def kernel(positions, neighbors):
    B, N, _ = positions.shape
    K = neighbors.shape[2]
    k = _make_sc_kernel(B, N, K, WPB=8, CA=125)
    out = k(positions[:, :, 0], positions[:, :, 1], positions[:, :, 2],
            neighbors.reshape(B, N * K))
    return out.reshape(B, N, K)


# speedup vs baseline: 494.6954x; 1.8591x over previous
import functools

import jax
import jax.numpy as jnp
from jax import lax
from jax.experimental import pallas as pl
from jax.experimental.pallas import tpu as pltpu
from jax.experimental.pallas import tpu_sc as plsc

L = 16


def _make_sc_kernel(B, N, K, WPB, SLAB, CA):
    NCH_FULL = SLAB // CA
    SLAB_LAST = N - (WPB - 1) * SLAB
    NCH_LAST = SLAB_LAST // CA
    KV = K // L
    info = plsc.get_sparse_core_info()
    NC = info.num_cores

    mesh = plsc.VectorSubcoreMesh(core_axis_name="c", subcore_axis_name="s")

    @functools.partial(
        pl.kernel,
        mesh=mesh,
        out_type=jax.ShapeDtypeStruct((B, N, K), jnp.float32),
        scratch_types=[
            pltpu.VMEM((N,), jnp.float32),
            pltpu.VMEM((N,), jnp.float32),
            pltpu.VMEM((N,), jnp.float32),
            pltpu.VMEM((2, CA, K), jnp.int32),
            pltpu.VMEM((2, CA, K), jnp.float32),
            pltpu.SemaphoreType.DMA((2,)),
            pltpu.SemaphoreType.DMA((2,)),
        ],
        compiler_params=pltpu.CompilerParams(
            use_tc_tiling_on_sc=True, needs_layout_passes=False
        ),
    )
    def k(xs_hbm, ys_hbm, zs_hbm, nbr_hbm, out_hbm,
          xs_v, ys_v, zs_v, nbr_v, out_v, in_sem, out_sem):
        wid = lax.axis_index("s") * NC + lax.axis_index("c")
        b = wid // WPB
        w = wid % WPB
        base_a = w * SLAB
        nch = jnp.where(w == WPB - 1, NCH_LAST, NCH_FULL)

        pltpu.sync_copy(xs_hbm.at[pl.ds(b * N, N)], xs_v)
        pltpu.sync_copy(ys_hbm.at[pl.ds(b * N, N)], ys_v)
        pltpu.sync_copy(zs_hbm.at[pl.ds(b * N, N)], zs_v)

        def in_copy(c, buf):
            a0 = base_a + c * CA
            return pltpu.make_async_copy(
                nbr_hbm.at[b, pl.ds(a0, CA)], nbr_v.at[buf], in_sem.at[buf])

        def out_copy(c, buf):
            a0 = base_a + c * CA
            return pltpu.make_async_copy(
                out_v.at[buf], out_hbm.at[b, pl.ds(a0, CA)], out_sem.at[buf])

        in_copy(0, 0).start()

        def chunk_body(c, carry):
            @pl.when(c < nch)
            def _():
                buf = lax.rem(c, 2)

                @pl.when(c >= 2)
                def _():
                    out_copy(c - 2, buf).wait()

                in_copy(c, buf).wait()

                @pl.when(c + 1 < nch)
                def _():
                    in_copy(c + 1, 1 - buf).start()

                a0c = base_a + c * CA

                def atom_body(a, carry2):
                    ag = a0c + a
                    agv = jnp.full((L,), ag, jnp.int32)
                    cx = plsc.load_gather(xs_v, [agv])
                    cy = plsc.load_gather(ys_v, [agv])
                    cz = plsc.load_gather(zs_v, [agv])
                    R = range(KV)
                    nb = [nbr_v[buf, a, pl.ds(L * j, L)] for j in R]
                    xg = [plsc.load_gather(xs_v, [nb[j]]) for j in R]
                    yg = [plsc.load_gather(ys_v, [nb[j]]) for j in R]
                    zg = [plsc.load_gather(zs_v, [nb[j]]) for j in R]
                    dx = [xg[j] - cx for j in R]
                    dy = [yg[j] - cy for j in R]
                    dz = [zg[j] - cz for j in R]
                    s = [dx[j] * dx[j] for j in R]
                    s = [s[j] + dy[j] * dy[j] for j in R]
                    s = [s[j] + dz[j] * dz[j] for j in R]
                    iv = [plsc.bitcast(s[j], jnp.int32) for j in R]
                    iv = [jnp.int32(0x5F375A86) - (iv[j] >> 1) for j in R]
                    y = [plsc.bitcast(iv[j], jnp.float32) for j in R]
                    xh = [s[j] * jnp.float32(0.5) for j in R]
                    t = [y[j] * y[j] for j in R]
                    t = [xh[j] * t[j] for j in R]
                    t = [jnp.float32(1.5) - t[j] for j in R]
                    y = [y[j] * t[j] for j in R]
                    d = [s[j] * y[j] for j in R]
                    for j in R:
                        out_v[buf, a, pl.ds(L * j, L)] = d[j]
                    return carry2

                lax.fori_loop(0, CA, atom_body, 0)
                out_copy(c, buf).start()
            return carry

        lax.fori_loop(0, NCH_FULL, chunk_body, 0)
        out_copy(nch - 2, lax.rem(nch - 2, 2)).wait()
        out_copy(nch - 1, lax.rem(nch - 1, 2)).wait()

    return k


def kernel(positions, neighbors):
    B, N, _ = positions.shape
    K = neighbors.shape[2]
    k = _make_sc_kernel(B, N, K, WPB=8, SLAB=3200, CA=40)
    return k(positions[:, :, 0].reshape(-1), positions[:, :, 1].reshape(-1),
             positions[:, :, 2].reshape(-1), neighbors)


# speedup vs baseline: 995.7675x; 2.0129x over previous
import functools

import jax
import jax.numpy as jnp
from jax import lax
from jax.experimental import pallas as pl
from jax.experimental.pallas import tpu as pltpu
from jax.experimental.pallas import tpu_sc as plsc

L = 16
U = 8


def _dists(nb, cx, cy, cz, xs_v, ys_v, zs_v):
    R = range(len(nb))
    xg = [plsc.load_gather(xs_v, [nb[j]]) for j in R]
    yg = [plsc.load_gather(ys_v, [nb[j]]) for j in R]
    zg = [plsc.load_gather(zs_v, [nb[j]]) for j in R]
    dx = [xg[j] - cx for j in R]
    dy = [yg[j] - cy for j in R]
    dz = [zg[j] - cz for j in R]
    s = [dx[j] * dx[j] for j in R]
    s = [s[j] + dy[j] * dy[j] for j in R]
    s = [s[j] + dz[j] * dz[j] for j in R]
    iv = [plsc.bitcast(s[j], jnp.int32) for j in R]
    iv = [jnp.int32(0x5F375A86) - (iv[j] >> 1) for j in R]
    y = [plsc.bitcast(iv[j], jnp.float32) for j in R]
    xh = [s[j] * jnp.float32(0.5) for j in R]
    t = [y[j] * y[j] for j in R]
    t = [xh[j] * t[j] for j in R]
    t = [jnp.float32(1.5) - t[j] for j in R]
    y = [y[j] * t[j] for j in R]
    return [s[j] * y[j] for j in R]


def _make_sc_kernel(B, N, K, WPB, CA):
    NCHUNK = N // CA
    TAIL = N - NCHUNK * CA
    N0 = NCHUNK * CA
    NG = CA // L
    KB = K // U
    NPAD = (NCHUNK + 1) * CA
    info = plsc.get_sparse_core_info()
    NC = info.num_cores

    mesh = plsc.VectorSubcoreMesh(core_axis_name="c", subcore_axis_name="s")

    @functools.partial(
        pl.kernel,
        mesh=mesh,
        out_type=jax.ShapeDtypeStruct((B, K, N), jnp.float32),
        scratch_types=[
            pltpu.VMEM((NPAD,), jnp.float32),
            pltpu.VMEM((NPAD,), jnp.float32),
            pltpu.VMEM((NPAD,), jnp.float32),
            pltpu.VMEM((2, K, CA), jnp.int32),
            pltpu.VMEM((2, K, CA), jnp.float32),
            pltpu.VMEM((K, TAIL), jnp.int32),
            pltpu.VMEM((K, TAIL), jnp.float32),
            pltpu.SemaphoreType.DMA((2,)),
            pltpu.SemaphoreType.DMA((2,)),
            pltpu.SemaphoreType.DMA,
        ],
        compiler_params=pltpu.CompilerParams(
            use_tc_tiling_on_sc=True, needs_layout_passes=False
        ),
    )
    def k(xs_hbm, ys_hbm, zs_hbm, nbr_hbm, out_hbm,
          xs_v, ys_v, zs_v, nbr_v, out_v, tnbr_v, tout_v,
          in_sem, out_sem, tail_sem):
        wid = lax.axis_index("s") * NC + lax.axis_index("c")
        b = wid // WPB
        w = wid % WPB
        nch = (NCHUNK - 1 - w) // WPB + 1

        pltpu.sync_copy(xs_hbm.at[pl.ds(b * N, N)], xs_v.at[pl.ds(0, N)])
        pltpu.sync_copy(ys_hbm.at[pl.ds(b * N, N)], ys_v.at[pl.ds(0, N)])
        pltpu.sync_copy(zs_hbm.at[pl.ds(b * N, N)], zs_v.at[pl.ds(0, N)])

        def in_copy(c, buf):
            return pltpu.make_async_copy(
                nbr_hbm.at[b, :, pl.ds(c * CA, CA)],
                nbr_v.at[buf], in_sem.at[buf])

        def out_copy(c, buf):
            return pltpu.make_async_copy(
                out_v.at[buf], out_hbm.at[b, :, pl.ds(c * CA, CA)],
                out_sem.at[buf])

        in_copy(w, 0).start()

        def chunk_body(t, carry):
            @pl.when(t < nch)
            def _():
                c = w + t * WPB
                buf = lax.rem(t, 2)

                @pl.when(t >= 2)
                def _():
                    out_copy(c - 2 * WPB, buf).wait()

                in_copy(c, buf).wait()

                @pl.when(t + 1 < nch)
                def _():
                    in_copy(c + WPB, 1 - buf).start()

                a0c = c * CA

                def group_body(g, carry2):
                    g16 = g * L
                    cx = xs_v[pl.ds(a0c + g16, L)]
                    cy = ys_v[pl.ds(a0c + g16, L)]
                    cz = zs_v[pl.ds(a0c + g16, L)]

                    def kb_body(kb, carry3):
                        k0 = kb * U
                        R = range(U)
                        nb = [nbr_v[buf, k0 + j, pl.ds(g16, L)] for j in R]
                        d = _dists(nb, cx, cy, cz, xs_v, ys_v, zs_v)
                        for j in R:
                            out_v[buf, k0 + j, pl.ds(g16, L)] = d[j]
                        return carry3

                    lax.fori_loop(0, KB, kb_body, 0)
                    return carry2

                lax.fori_loop(0, NG, group_body, 0)
                out_copy(c, buf).start()
            return carry

        lax.fori_loop(0, (NCHUNK - 1) // WPB + 1, chunk_body, 0)

        if TAIL:
            @pl.when(w == WPB - 1)
            def _():
                iota = lax.iota(jnp.int32, L)
                pltpu.make_async_copy(
                    nbr_hbm.at[b, :, pl.ds(N0, TAIL)], tnbr_v,
                    tail_sem).start()
                pltpu.make_async_copy(
                    nbr_hbm.at[b, :, pl.ds(N0, TAIL)], tnbr_v,
                    tail_sem).wait()
                for g in range((TAIL + L - 1) // L):
                    al = jnp.minimum(g * L + iota, TAIL - 1)
                    cidx = N0 + al
                    cx = plsc.load_gather(xs_v, [cidx])
                    cy = plsc.load_gather(ys_v, [cidx])
                    cz = plsc.load_gather(zs_v, [cidx])
                    for k0 in range(0, K, U):
                        R = range(U)
                        kv = [jnp.full((L,), k0 + j, jnp.int32) for j in R]
                        nb = [plsc.load_gather(tnbr_v, [kv[j], al]) for j in R]
                        d = _dists(nb, cx, cy, cz, xs_v, ys_v, zs_v)
                        for j in R:
                            plsc.store_scatter(tout_v, [kv[j], al], d[j])
                pltpu.make_async_copy(
                    tout_v, out_hbm.at[b, :, pl.ds(N0, TAIL)],
                    tail_sem).start()
                pltpu.make_async_copy(
                    tout_v, out_hbm.at[b, :, pl.ds(N0, TAIL)],
                    tail_sem).wait()

        out_copy(w + (nch - 2) * WPB, lax.rem(nch - 2, 2)).wait()
        out_copy(w + (nch - 1) * WPB, lax.rem(nch - 1, 2)).wait()

    return k


def kernel(positions, neighbors):
    B, N, _ = positions.shape
    K = neighbors.shape[2]
    k = _make_sc_kernel(B, N, K, WPB=8, CA=128)
    out = k(positions[:, :, 0].reshape(-1), positions[:, :, 1].reshape(-1),
            positions[:, :, 2].reshape(-1), jnp.swapaxes(neighbors, 1, 2))
    return jnp.swapaxes(out, 1, 2)


# speedup vs baseline: 1112.3111x; 1.1170x over previous
import functools

import jax
import jax.numpy as jnp
from jax import lax
from jax.experimental import pallas as pl
from jax.experimental.pallas import tpu as pltpu
from jax.experimental.pallas import tpu_sc as plsc

L = 16
U = 8


def _dists(nb, cx, cy, cz, xs_v, ys_v, zs_v):
    R = range(len(nb))
    xg = [plsc.load_gather(xs_v, [nb[j]]) for j in R]
    yg = [plsc.load_gather(ys_v, [nb[j]]) for j in R]
    zg = [plsc.load_gather(zs_v, [nb[j]]) for j in R]
    dx = [xg[j] - cx for j in R]
    dy = [yg[j] - cy for j in R]
    dz = [zg[j] - cz for j in R]
    s = [dx[j] * dx[j] for j in R]
    s = [s[j] + dy[j] * dy[j] for j in R]
    s = [s[j] + dz[j] * dz[j] for j in R]
    iv = [plsc.bitcast(s[j], jnp.int32) for j in R]
    iv = [jnp.int32(0x5F375A86) - (iv[j] >> 1) for j in R]
    y = [plsc.bitcast(iv[j], jnp.float32) for j in R]
    xh = [s[j] * jnp.float32(0.5) for j in R]
    t = [y[j] * y[j] for j in R]
    t = [xh[j] * t[j] for j in R]
    t = [jnp.float32(1.5) - t[j] for j in R]
    y = [y[j] * t[j] for j in R]
    return [s[j] * y[j] for j in R]


def _make_sc_kernel(B, N, K, WPB, CA):
    NCHUNK = N // CA
    TAIL = N - NCHUNK * CA
    N0 = NCHUNK * CA
    NG = CA // L
    KB = K // U
    NPAD = (NCHUNK + 1) * CA
    info = plsc.get_sparse_core_info()
    NC = info.num_cores

    mesh = plsc.VectorSubcoreMesh(core_axis_name="c", subcore_axis_name="s")

    @functools.partial(
        pl.kernel,
        mesh=mesh,
        out_type=jax.ShapeDtypeStruct((B, K, N), jnp.float32),
        scratch_types=[
            pltpu.VMEM((NPAD,), jnp.float32),
            pltpu.VMEM((NPAD,), jnp.float32),
            pltpu.VMEM((NPAD,), jnp.float32),
            pltpu.VMEM((2, K, CA), jnp.int32),
            pltpu.VMEM((2, K, CA), jnp.float32),
            pltpu.VMEM((K, TAIL), jnp.int32),
            pltpu.VMEM((K, TAIL), jnp.float32),
            pltpu.SemaphoreType.DMA((2,)),
            pltpu.SemaphoreType.DMA((2,)),
            pltpu.SemaphoreType.DMA,
        ],
        compiler_params=pltpu.CompilerParams(
            use_tc_tiling_on_sc=True, needs_layout_passes=False
        ),
    )
    def k(xs_hbm, ys_hbm, zs_hbm, nbr_hbm, out_hbm,
          xs_v, ys_v, zs_v, nbr_v, out_v, tnbr_v, tout_v,
          in_sem, out_sem, tail_sem):
        wid = lax.axis_index("s") * NC + lax.axis_index("c")
        b = wid // WPB
        w = wid % WPB
        nch = (NCHUNK - 1 - w) // WPB + 1

        pltpu.sync_copy(xs_hbm.at[pl.ds(b * N, N)], xs_v.at[pl.ds(0, N)])
        pltpu.sync_copy(ys_hbm.at[pl.ds(b * N, N)], ys_v.at[pl.ds(0, N)])
        pltpu.sync_copy(zs_hbm.at[pl.ds(b * N, N)], zs_v.at[pl.ds(0, N)])

        def in_copy(c, buf):
            return pltpu.make_async_copy(
                nbr_hbm.at[b, :, pl.ds(c * CA, CA)],
                nbr_v.at[buf], in_sem.at[buf])

        def out_copy(c, buf):
            return pltpu.make_async_copy(
                out_v.at[buf], out_hbm.at[b, :, pl.ds(c * CA, CA)],
                out_sem.at[buf])

        in_copy(w, 0).start()

        def chunk_body(t, carry):
            @pl.when(t < nch)
            def _():
                c = w + t * WPB
                buf = lax.rem(t, 2)

                @pl.when(t >= 2)
                def _():
                    out_copy(c - 2 * WPB, buf).wait()

                in_copy(c, buf).wait()

                @pl.when(t + 1 < nch)
                def _():
                    in_copy(c + WPB, 1 - buf).start()

                a0c = c * CA

                def group_body(g, carry2):
                    g16 = g * L
                    cx = xs_v[pl.ds(a0c + g16, L)]
                    cy = ys_v[pl.ds(a0c + g16, L)]
                    cz = zs_v[pl.ds(a0c + g16, L)]

                    @plsc.parallel_loop(0, KB, 1, unroll=2)
                    def _(kb):
                        k0 = kb * U
                        R = range(U)
                        nb = [nbr_v[buf, k0 + j, pl.ds(g16, L)] for j in R]
                        d = _dists(nb, cx, cy, cz, xs_v, ys_v, zs_v)
                        for j in R:
                            out_v[buf, k0 + j, pl.ds(g16, L)] = d[j]
                    return carry2

                lax.fori_loop(0, NG, group_body, 0)
                out_copy(c, buf).start()
            return carry

        lax.fori_loop(0, (NCHUNK - 1) // WPB + 1, chunk_body, 0)

        if TAIL:
            @pl.when(w == WPB - 1)
            def _():
                iota = lax.iota(jnp.int32, L)
                pltpu.make_async_copy(
                    nbr_hbm.at[b, :, pl.ds(N0, TAIL)], tnbr_v,
                    tail_sem).start()
                pltpu.make_async_copy(
                    nbr_hbm.at[b, :, pl.ds(N0, TAIL)], tnbr_v,
                    tail_sem).wait()
                for g in range((TAIL + L - 1) // L):
                    al = jnp.minimum(g * L + iota, TAIL - 1)
                    cidx = N0 + al
                    cx = plsc.load_gather(xs_v, [cidx])
                    cy = plsc.load_gather(ys_v, [cidx])
                    cz = plsc.load_gather(zs_v, [cidx])
                    for k0 in range(0, K, U):
                        R = range(U)
                        kv = [jnp.full((L,), k0 + j, jnp.int32) for j in R]
                        nb = [plsc.load_gather(tnbr_v, [kv[j], al]) for j in R]
                        d = _dists(nb, cx, cy, cz, xs_v, ys_v, zs_v)
                        for j in R:
                            plsc.store_scatter(tout_v, [kv[j], al], d[j])
                pltpu.make_async_copy(
                    tout_v, out_hbm.at[b, :, pl.ds(N0, TAIL)],
                    tail_sem).start()
                pltpu.make_async_copy(
                    tout_v, out_hbm.at[b, :, pl.ds(N0, TAIL)],
                    tail_sem).wait()

        out_copy(w + (nch - 2) * WPB, lax.rem(nch - 2, 2)).wait()
        out_copy(w + (nch - 1) * WPB, lax.rem(nch - 1, 2)).wait()

    return k


def kernel(positions, neighbors):
    B, N, _ = positions.shape
    K = neighbors.shape[2]
    k = _make_sc_kernel(B, N, K, WPB=8, CA=128)
    out = k(positions[:, :, 0].reshape(-1), positions[:, :, 1].reshape(-1),
            positions[:, :, 2].reshape(-1), jnp.swapaxes(neighbors, 1, 2))
    return jnp.swapaxes(out, 1, 2)
